# baseline (speedup 1.0000x reference)
import jax
import jax.numpy as jnp
from jax import lax
from jax.experimental import pallas as pl
from jax.experimental.pallas import tpu as pltpu
from jax.experimental.pallas import tpu_sc as plsc

C = 128
LANES = 16
NSUB = 16
NTOK = 16384
TPW = NTOK // NSUB
RPW = 8
COLS = TPW // RPW
VPR = COLS // LANES
NBIN = LANES * C
DEAD = NBIN
MROWS = NTOK // COLS


def _wce_body(preds_hbm, ml_hbm, out_hbm,
              ml_v, idx_v, bidx_v, g_v, cntb, psumb, part_v,
              allp_v, out_v, shared, sem, sem2):
    w = lax.axis_index("s")
    base = w * TPW
    pltpu.async_copy(ml_hbm.at[pl.ds(w * RPW, RPW)], ml_v, sem2).wait()

    iota = lax.iota(jnp.int32, LANES)
    lane_row = iota * C
    dead = DEAD + iota
    zerov = jnp.zeros((LANES,), jnp.float32)
    onev = jnp.ones((LANES,), jnp.float32)

    gcopies = []
    for r in range(RPW):
        rbase = (base + r * COLS) * C
        for k in range(VPR):
            sl = pl.ds(k * LANES, LANES)
            ml = ml_v[r, sl]
            live = lane_row + (ml & (C - 1))
            bidx_v[r, sl] = jnp.where(ml >= 256, live, dead)
            idx_v[r, sl] = rbase + k * (LANES * C) + live
        gcopies.append(pltpu.async_copy(preds_hbm.at[idx_v.at[r]], g_v.at[r], sem))

    for i in range(NBIN // LANES):
        cntb[pl.ds(i * LANES, LANES)] = zerov
        psumb[pl.ds(i * LANES, LANES)] = zerov

    for r in range(RPW):
        gcopies[r].wait()
        for k in range(VPR):
            sl = pl.ds(k * LANES, LANES)
            bidx = bidx_v[r, sl]
            g = g_v[r, sl]
            plsc.addupdate_scatter(cntb, [bidx], onev)
            plsc.addupdate_scatter(psumb, [bidx], g)

    for k in range(C // LANES):
        sl = pl.ds(k * LANES, LANES)
        acs = [cntb[pl.ds(l * C + k * LANES, LANES)] for l in range(LANES)]
        aps = [psumb[pl.ds(l * C + k * LANES, LANES)] for l in range(LANES)]
        while len(acs) > 1:
            acs = [acs[i] + acs[i + 1] for i in range(0, len(acs), 2)]
            aps = [aps[i] + aps[i + 1] for i in range(0, len(aps), 2)]
        part_v[sl] = acs[0]
        part_v[pl.ds(C + k * LANES, LANES)] = aps[0]

    pltpu.sync_copy(part_v, shared.at[w])
    plsc.subcore_barrier()

    @pl.when(w == 0)
    def _final():
        pltpu.sync_copy(shared, allp_v)
        cnt, ps = [], []
        for k in range(C // LANES):
            acs = [allp_v[t, pl.ds(k * LANES, LANES)] for t in range(NSUB)]
            aps = [allp_v[t, pl.ds(C + k * LANES, LANES)] for t in range(NSUB)]
            while len(acs) > 1:
                acs = [acs[i] + acs[i + 1] for i in range(0, len(acs), 2)]
                aps = [aps[i] + aps[i + 1] for i in range(0, len(aps), 2)]
            cnt.append(acs[0])
            ps.append(aps[0])
        mv = cnt[0]
        for k in range(1, C // LANES):
            mv = jnp.minimum(mv, cnt[k])
        mmin = jnp.min(mv)
        num = jnp.zeros((LANES,), jnp.float32)
        den = jnp.zeros((LANES,), jnp.float32)
        for k in range(C // LANES):
            wgt = mmin / (cnt[k] + 1e-8)
            num = num + wgt * ps[k]
            den = den + wgt * cnt[k]
        numv = jnp.full((LANES,), jnp.sum(num), jnp.float32)
        denv = jnp.full((LANES,), jnp.sum(den), jnp.float32)
        out_v[...] = -(numv / denv)
        pltpu.sync_copy(out_v, out_hbm)


def kernel(preds, labels, pad_mask):
    b, s, c = preds.shape
    preds_f = preds.reshape(b * s * c)
    ml = (labels.astype(jnp.int32)
          | (pad_mask.astype(jnp.int32) << 8)).reshape(MROWS, COLS)
    mesh = plsc.VectorSubcoreMesh(
        core_axis_name="c", subcore_axis_name="s", num_cores=1)
    out = pl.kernel(
        _wce_body,
        out_type=jax.ShapeDtypeStruct((LANES,), jnp.float32),
        mesh=mesh,
        compiler_params=pltpu.CompilerParams(needs_layout_passes=False),
        scratch_types=[
            pltpu.VMEM((RPW, COLS), jnp.int32),
            pltpu.VMEM((RPW, COLS), jnp.int32),
            pltpu.VMEM((RPW, COLS), jnp.int32),
            pltpu.VMEM((RPW, COLS), jnp.float32),
            pltpu.VMEM((NBIN + LANES,), jnp.float32),
            pltpu.VMEM((NBIN + LANES,), jnp.float32),
            pltpu.VMEM((2 * C,), jnp.float32),
            pltpu.VMEM((NSUB, 2 * C), jnp.float32),
            pltpu.VMEM((LANES,), jnp.float32),
            pltpu.VMEM_SHARED((NSUB, 2 * C), jnp.float32),
            pltpu.SemaphoreType.DMA,
            pltpu.SemaphoreType.DMA,
        ],
    )(preds_f, ml)
    return out[0]
